# Initial kernel scaffold; baseline (speedup 1.0000x reference)
import jax
import jax.numpy as jnp
from jax.experimental import pallas as pl


def kernel(x, pos_enc, params, edge_index, node_indices):
    raise NotImplementedError("write your pallas kernel here")


# speedup vs baseline: 2.6117x; 2.6117x over previous
import functools

import jax
import jax.numpy as jnp
from jax import lax
from jax.experimental import pallas as pl
from jax.experimental.pallas import tpu as pltpu
from jax.experimental.pallas import tpu_sc as plsc

N = 10000
E = 320000
D = 128
POS_DIM = 4
FFN_DIM = 256

NC = 2
NS = 16
NW = NC * NS
EPT = E // NW
BLK = 80
NBLK = EPT // BLK
RPT8 = (N // NS) // 8 * 8
RPT_TAIL = N - NS * RPT8
NCHUNK = D // 16

_sc_mesh = plsc.VectorSubcoreMesh(core_axis_name="c", subcore_axis_name="s")


@functools.partial(
    pl.kernel,
    out_type=jax.ShapeDtypeStruct((NC, N, D), jnp.float32),
    mesh=_sc_mesh,
    scratch_types=[
        pltpu.VMEM((BLK,), jnp.int32),
        pltpu.VMEM((BLK,), jnp.int32),
        pltpu.VMEM((BLK, D), jnp.float32),
        pltpu.VMEM((BLK, 16), jnp.float32),
        pltpu.VMEM((BLK, 16), jnp.float32),
        pltpu.VMEM((BLK, D), jnp.float32),
        pltpu.VMEM((POS_DIM, D), jnp.float32),
        pltpu.VMEM((D,), jnp.float32),
        pltpu.VMEM_SHARED((N, D), jnp.float32),
        pltpu.SemaphoreType.DMA,
    ],
    compiler_params=pltpu.CompilerParams(use_tc_tiling_on_sc=False),
)
def _mp_kernel(x_hbm, pos16_hbm, src_hbm, dst_hbm, wc_hbm, bc_hbm, zeros_hbm,
               out_hbm,
               src_v, dst_v, xr_v, ps_v, pd_v, m_v, wc_v, bc_v, agg_sh, sem):
    ci = lax.axis_index("c")
    sid = lax.axis_index("s")
    wid = ci * NS + sid

    pltpu.sync_copy(zeros_hbm.at[pl.ds(sid * RPT8, RPT8)],
                    agg_sh.at[pl.ds(sid * RPT8, RPT8)])

    @pl.when(sid == NS - 1)
    def _zero_tail():
        pltpu.sync_copy(zeros_hbm.at[pl.ds(NS * RPT8, RPT_TAIL)],
                        agg_sh.at[pl.ds(NS * RPT8, RPT_TAIL)])
    pltpu.sync_copy(wc_hbm, wc_v)
    pltpu.sync_copy(bc_hbm, bc_v)
    plsc.subcore_barrier()

    wcs = [[wc_v[i, pl.ds(c * 16, 16)] for c in range(NCHUNK)]
           for i in range(POS_DIM)]
    bcs = [bc_v[pl.ds(c * 16, 16)] for c in range(NCHUNK)]

    ebase = wid * EPT

    def block_body(b, _):
        base = ebase + b * BLK
        pltpu.sync_copy(src_hbm.at[pl.ds(base, BLK)], src_v)
        pltpu.sync_copy(dst_hbm.at[pl.ds(base, BLK)], dst_v)
        c1 = pltpu.async_copy(x_hbm.at[src_v], xr_v, sem)
        c2 = pltpu.async_copy(pos16_hbm.at[src_v], ps_v, sem)
        c3 = pltpu.async_copy(pos16_hbm.at[dst_v], pd_v, sem)
        c1.wait()
        c2.wait()
        c3.wait()

        def edge_body(e, _):
            dr = jnp.abs(ps_v[e, :] - pd_v[e, :])
            d = [dr[i] for i in range(POS_DIM)]
            for c in range(NCHUNK):
                acc = xr_v[e, pl.ds(c * 16, 16)] + bcs[c]
                for i in range(POS_DIM):
                    acc = acc + d[i] * wcs[i][c]
                m_v[e, pl.ds(c * 16, 16)] = jnp.maximum(acc, 0.0)
            return 0

        lax.fori_loop(0, BLK, edge_body, 0)
        pltpu.sync_copy(m_v, agg_sh.at[dst_v], add=True)
        return 0

    lax.fori_loop(0, NBLK, block_body, 0)
    plsc.subcore_barrier()
    pltpu.sync_copy(agg_sh.at[pl.ds(sid * RPT8, RPT8)],
                    out_hbm.at[ci, pl.ds(sid * RPT8, RPT8)])

    @pl.when(sid == NS - 1)
    def _write_tail():
        pltpu.sync_copy(agg_sh.at[pl.ds(NS * RPT8, RPT_TAIL)],
                        out_hbm.at[ci, pl.ds(NS * RPT8, RPT_TAIL)])


_ROWS = 1000


def _dense_body(scale_ref, x_ref, a0_ref, a1_ref, pos_ref, pad_ref,
                w1_ref, b1_ref, w2_ref, b2_ref, g1_ref, be1_ref,
                wf1_ref, bf1_ref, wf2_ref, bf2_ref, g2_ref, be2_ref,
                offw_ref, offb_ref,
                xo_ref, poso_ref, pado_ref):
    x = x_ref[...]
    h = scale_ref[0, 0] * x + (a0_ref[...] + a1_ref[...])
    h = jnp.maximum(jnp.dot(h, w1_ref[...], preferred_element_type=jnp.float32)
                    + b1_ref[...], 0.0)
    h = jnp.dot(h, w2_ref[...], preferred_element_type=jnp.float32) + b2_ref[...]
    h = x + h
    mu = jnp.mean(h, axis=-1, keepdims=True)
    var = jnp.mean(jnp.square(h - mu), axis=-1, keepdims=True)
    h = (h - mu) * lax.rsqrt(var + 1e-5) * g1_ref[...] + be1_ref[...]
    f = jnp.maximum(jnp.dot(h, wf1_ref[...], preferred_element_type=jnp.float32)
                    + bf1_ref[...], 0.0)
    f = jnp.dot(f, wf2_ref[...], preferred_element_type=jnp.float32) + bf2_ref[...]
    h = h + f
    mu = jnp.mean(h, axis=-1, keepdims=True)
    var = jnp.mean(jnp.square(h - mu), axis=-1, keepdims=True)
    h = (h - mu) * lax.rsqrt(var + 1e-5) * g2_ref[...] + be2_ref[...]
    xo_ref[...] = h
    off = jnp.dot(h, offw_ref[...], preferred_element_type=jnp.float32) + offb_ref[...]
    poso_ref[...] = pos_ref[...] + off
    pado_ref[...] = pad_ref[...] + off


def _dense_layer(scale, x, agg, pos, pad, p, off_w, off_b):
    row_spec = pl.BlockSpec((_ROWS, D), lambda i: (i, 0))
    pos_spec = pl.BlockSpec((_ROWS, POS_DIM), lambda i: (i, 0))

    def full(a):
        return pl.BlockSpec(a.shape, lambda i: tuple(0 for _ in a.shape))

    w1 = p["W1"]
    b1 = p["b1"].reshape(1, D)
    w2 = p["W2"]
    b2 = p["b2"].reshape(1, D)
    g1 = p["ln1_g"].reshape(1, D)
    be1 = p["ln1_b"].reshape(1, D)
    wf1 = p["Wf1"]
    bf1 = p["bf1"].reshape(1, FFN_DIM)
    wf2 = p["Wf2"]
    bf2 = p["bf2"].reshape(1, D)
    g2 = p["ln2_g"].reshape(1, D)
    be2 = p["ln2_b"].reshape(1, D)
    offb = off_b.reshape(1, POS_DIM)
    scale2 = scale.reshape(1, 1)

    return pl.pallas_call(
        _dense_body,
        grid=(N // _ROWS,),
        in_specs=[
            full(scale2), row_spec, row_spec, row_spec, pos_spec, pos_spec,
            full(w1), full(b1), full(w2), full(b2), full(g1), full(be1),
            full(wf1), full(bf1), full(wf2), full(bf2), full(g2), full(be2),
            full(off_w), full(offb),
        ],
        out_specs=[row_spec, pos_spec, pos_spec],
        out_shape=[
            jax.ShapeDtypeStruct((N, D), jnp.float32),
            jax.ShapeDtypeStruct((N, POS_DIM), jnp.float32),
            jax.ShapeDtypeStruct((N, POS_DIM), jnp.float32),
        ],
    )(scale2, x, agg[0], agg[1], pos, pad,
      w1, b1, w2, b2, g1, be1, wf1, bf1, wf2, bf2, g2, be2, off_w, offb)


def kernel(x, pos_enc, params, edge_index, node_indices):
    src = edge_index[0].astype(jnp.int32)
    dst = edge_index[1].astype(jnp.int32)
    we = params["edge_embed_W"]
    be = params["edge_embed_b"]
    off_w = params["off_W"]
    off_b = params["off_b"]
    zeros = jnp.zeros((N, D), jnp.float32)

    pos = pos_enc
    pad = jnp.zeros((N, POS_DIM), jnp.float32)
    for p in params["layers"]:
        wc = we @ p["edge_W"]
        bc = be @ p["edge_W"] + p["edge_b"]
        pos16 = jnp.pad(pos, ((0, 0), (0, 16 - POS_DIM)))
        agg = _mp_kernel(x, pos16, src, dst, wc, bc, zeros)
        scale = (1.0 + p["eps"]).astype(jnp.float32)
        x, pos, pad = _dense_layer(scale, x, agg, pos, pad, p, off_w, off_b)
    return jnp.concatenate((pad, x), axis=1)


# speedup vs baseline: 3.1464x; 1.2048x over previous
import functools

import jax
import jax.numpy as jnp
from jax import lax
from jax.experimental import pallas as pl
from jax.experimental.pallas import tpu as pltpu
from jax.experimental.pallas import tpu_sc as plsc

N = 10000
E = 320000
D = 128
POS_DIM = 4
FFN_DIM = 256

NC = 2
NS = 16
NW = NC * NS
EPT = E // NW
BLK = 100
NBLK = EPT // BLK
RPT8 = (N // NS) // 8 * 8
RPT_TAIL = N - NS * RPT8
NCHUNK = D // 16
UNROLL = 5

_sc_mesh = plsc.VectorSubcoreMesh(core_axis_name="c", subcore_axis_name="s")


@functools.partial(
    pl.kernel,
    out_type=jax.ShapeDtypeStruct((NC, N, D), jnp.float32),
    mesh=_sc_mesh,
    scratch_types=[
        pltpu.VMEM((BLK,), jnp.int32),
        pltpu.VMEM((BLK,), jnp.int32),
        pltpu.VMEM((BLK,), jnp.int32),
        pltpu.VMEM((BLK,), jnp.int32),
        pltpu.VMEM((BLK, D), jnp.float32),
        pltpu.VMEM((BLK, D), jnp.float32),
        pltpu.VMEM((BLK, 16), jnp.float32),
        pltpu.VMEM((BLK, 16), jnp.float32),
        pltpu.VMEM((BLK, 16), jnp.float32),
        pltpu.VMEM((BLK, 16), jnp.float32),
        pltpu.VMEM((BLK, D), jnp.float32),
        pltpu.VMEM((POS_DIM, D), jnp.float32),
        pltpu.VMEM((D,), jnp.float32),
        pltpu.VMEM_SHARED((N, D), jnp.float32),
        pltpu.SemaphoreType.DMA,
        pltpu.SemaphoreType.DMA,
        pltpu.SemaphoreType.DMA,
        pltpu.SemaphoreType.DMA,
    ],
    compiler_params=pltpu.CompilerParams(use_tc_tiling_on_sc=False),
)
def _mp_kernel(x_hbm, pos16_hbm, src_hbm, dst_hbm, wc_hbm, bc_hbm, zeros_hbm,
               out_hbm,
               src0_v, src1_v, dst0_v, dst1_v, xr0_v, xr1_v,
               ps0_v, ps1_v, pd0_v, pd1_v,
               m_v, wc_v, bc_v, agg_sh, sem_ia, sem_ib, sem_ga, sem_gb):
    ci = lax.axis_index("c")
    sid = lax.axis_index("s")
    wid = ci * NS + sid

    pltpu.sync_copy(zeros_hbm.at[pl.ds(sid * RPT8, RPT8)],
                    agg_sh.at[pl.ds(sid * RPT8, RPT8)])

    @pl.when(sid == NS - 1)
    def _zero_tail():
        pltpu.sync_copy(zeros_hbm.at[pl.ds(NS * RPT8, RPT_TAIL)],
                        agg_sh.at[pl.ds(NS * RPT8, RPT_TAIL)])
    pltpu.sync_copy(wc_hbm, wc_v)
    pltpu.sync_copy(bc_hbm, bc_v)
    plsc.subcore_barrier()

    wcs = [[wc_v[i, pl.ds(c * 16, 16)] for c in range(NCHUNK)]
           for i in range(POS_DIM)]
    bcs = [bc_v[pl.ds(c * 16, 16)] for c in range(NCHUNK)]

    bufs = [
        (src0_v, dst0_v, xr0_v, ps0_v, pd0_v, sem_ia, sem_ga),
        (src1_v, dst1_v, xr1_v, ps1_v, pd1_v, sem_ib, sem_gb),
    ]

    def fire_idx(b, p):
        src_v, dst_v, _, _, _, sem_i, _ = bufs[p]
        pltpu.async_copy(src_hbm.at[wid, b], src_v, sem_i)
        pltpu.async_copy(dst_hbm.at[wid, b], dst_v, sem_i)

    def drain_idx(b, p):
        src_v, dst_v, _, _, _, sem_i, _ = bufs[p]
        pltpu.make_async_copy(src_hbm.at[wid, b], src_v, sem_i).wait()
        pltpu.make_async_copy(dst_hbm.at[wid, b], dst_v, sem_i).wait()

    def fire_gather(p):
        src_v, dst_v, xr, ps, pd, _, sem_g = bufs[p]
        pltpu.async_copy(x_hbm.at[src_v], xr, sem_g)
        pltpu.async_copy(pos16_hbm.at[src_v], ps, sem_g)
        pltpu.async_copy(pos16_hbm.at[dst_v], pd, sem_g)

    def drain_gather(p):
        src_v, dst_v, xr, ps, pd, _, sem_g = bufs[p]
        pltpu.make_async_copy(x_hbm.at[src_v], xr, sem_g).wait()
        pltpu.make_async_copy(pos16_hbm.at[src_v], ps, sem_g).wait()
        pltpu.make_async_copy(pos16_hbm.at[dst_v], pd, sem_g).wait()

    def compute(p):
        src_v, dst_v, xr, ps, pd, _, _ = bufs[p]

        def edge_body(g, _):
            for l in range(UNROLL):
                e = g * UNROLL + l
                dr = jnp.abs(ps[e, :] - pd[e, :])
                d = [dr[i] for i in range(POS_DIM)]
                for c in range(NCHUNK):
                    acc = xr[e, pl.ds(c * 16, 16)] + bcs[c]
                    for i in range(POS_DIM):
                        acc = acc + d[i] * wcs[i][c]
                    m_v[e, pl.ds(c * 16, 16)] = jnp.maximum(acc, 0.0)
            return 0

        lax.fori_loop(0, BLK // UNROLL, edge_body, 0)
        pltpu.sync_copy(m_v, agg_sh.at[dst_v], add=True)

    fire_idx(0, 0)
    drain_idx(0, 0)
    fire_gather(0)
    fire_idx(1, 1)

    def two_blocks(t, _):
        b1 = 2 * t + 1
        drain_idx(b1, 1)
        fire_gather(1)
        drain_gather(0)
        compute(0)

        @pl.when(b1 + 1 < NBLK)
        def _prep_even():
            fire_idx(b1 + 1, 0)
            drain_idx(b1 + 1, 0)
            fire_gather(0)

        drain_gather(1)
        compute(1)

        @pl.when(b1 + 2 < NBLK)
        def _prep_odd():
            fire_idx(b1 + 2, 1)

        return 0

    lax.fori_loop(0, NBLK // 2, two_blocks, 0)
    plsc.subcore_barrier()
    pltpu.sync_copy(agg_sh.at[pl.ds(sid * RPT8, RPT8)],
                    out_hbm.at[ci, pl.ds(sid * RPT8, RPT8)])

    @pl.when(sid == NS - 1)
    def _write_tail():
        pltpu.sync_copy(agg_sh.at[pl.ds(NS * RPT8, RPT_TAIL)],
                        out_hbm.at[ci, pl.ds(NS * RPT8, RPT_TAIL)])


_ROWS = 1000


def _dense_body(scale_ref, x_ref, a0_ref, a1_ref, pos_ref, pad_ref,
                w1_ref, b1_ref, w2_ref, b2_ref, g1_ref, be1_ref,
                wf1_ref, bf1_ref, wf2_ref, bf2_ref, g2_ref, be2_ref,
                offw_ref, offb_ref,
                xo_ref, poso_ref, pado_ref):
    x = x_ref[...]
    h = scale_ref[0, 0] * x + (a0_ref[...] + a1_ref[...])
    h = jnp.maximum(jnp.dot(h, w1_ref[...], preferred_element_type=jnp.float32)
                    + b1_ref[...], 0.0)
    h = jnp.dot(h, w2_ref[...], preferred_element_type=jnp.float32) + b2_ref[...]
    h = x + h
    mu = jnp.mean(h, axis=-1, keepdims=True)
    var = jnp.mean(jnp.square(h - mu), axis=-1, keepdims=True)
    h = (h - mu) * lax.rsqrt(var + 1e-5) * g1_ref[...] + be1_ref[...]
    f = jnp.maximum(jnp.dot(h, wf1_ref[...], preferred_element_type=jnp.float32)
                    + bf1_ref[...], 0.0)
    f = jnp.dot(f, wf2_ref[...], preferred_element_type=jnp.float32) + bf2_ref[...]
    h = h + f
    mu = jnp.mean(h, axis=-1, keepdims=True)
    var = jnp.mean(jnp.square(h - mu), axis=-1, keepdims=True)
    h = (h - mu) * lax.rsqrt(var + 1e-5) * g2_ref[...] + be2_ref[...]
    xo_ref[...] = h
    off = jnp.dot(h, offw_ref[...], preferred_element_type=jnp.float32) + offb_ref[...]
    poso_ref[...] = pos_ref[...] + off
    pado_ref[...] = pad_ref[...] + off


def _dense_layer(scale, x, agg, pos, pad, p, off_w, off_b):
    row_spec = pl.BlockSpec((_ROWS, D), lambda i: (i, 0))
    pos_spec = pl.BlockSpec((_ROWS, POS_DIM), lambda i: (i, 0))

    def full(a):
        return pl.BlockSpec(a.shape, lambda i: tuple(0 for _ in a.shape))

    w1 = p["W1"]
    b1 = p["b1"].reshape(1, D)
    w2 = p["W2"]
    b2 = p["b2"].reshape(1, D)
    g1 = p["ln1_g"].reshape(1, D)
    be1 = p["ln1_b"].reshape(1, D)
    wf1 = p["Wf1"]
    bf1 = p["bf1"].reshape(1, FFN_DIM)
    wf2 = p["Wf2"]
    bf2 = p["bf2"].reshape(1, D)
    g2 = p["ln2_g"].reshape(1, D)
    be2 = p["ln2_b"].reshape(1, D)
    offb = off_b.reshape(1, POS_DIM)
    scale2 = scale.reshape(1, 1)

    return pl.pallas_call(
        _dense_body,
        grid=(N // _ROWS,),
        in_specs=[
            full(scale2), row_spec, row_spec, row_spec, pos_spec, pos_spec,
            full(w1), full(b1), full(w2), full(b2), full(g1), full(be1),
            full(wf1), full(bf1), full(wf2), full(bf2), full(g2), full(be2),
            full(off_w), full(offb),
        ],
        out_specs=[row_spec, pos_spec, pos_spec],
        out_shape=[
            jax.ShapeDtypeStruct((N, D), jnp.float32),
            jax.ShapeDtypeStruct((N, POS_DIM), jnp.float32),
            jax.ShapeDtypeStruct((N, POS_DIM), jnp.float32),
        ],
    )(scale2, x, agg[0], agg[1], pos, pad,
      w1, b1, w2, b2, g1, be1, wf1, bf1, wf2, bf2, g2, be2, off_w, offb)


def kernel(x, pos_enc, params, edge_index, node_indices):
    src = edge_index[0].astype(jnp.int32)
    dst = edge_index[1].astype(jnp.int32)
    we = params["edge_embed_W"]
    be = params["edge_embed_b"]
    off_w = params["off_W"]
    off_b = params["off_b"]
    zeros = jnp.zeros((N, D), jnp.float32)

    pos = pos_enc
    pad = jnp.zeros((N, POS_DIM), jnp.float32)
    for p in params["layers"]:
        wc = we @ p["edge_W"]
        bc = be @ p["edge_W"] + p["edge_b"]
        pos16 = jnp.pad(pos, ((0, 0), (0, 16 - POS_DIM)))
        agg = _mp_kernel(x, pos16, src.reshape(NW, NBLK, BLK),
                         dst.reshape(NW, NBLK, BLK), wc, bc, zeros)
        scale = (1.0 + p["eps"]).astype(jnp.float32)
        x, pos, pad = _dense_layer(scale, x, agg, pos, pad, p, off_w, off_b)
    return jnp.concatenate((pad, x), axis=1)


# speedup vs baseline: 3.7560x; 1.1937x over previous
import functools

import jax
import jax.numpy as jnp
from jax import lax
from jax.experimental import pallas as pl
from jax.experimental.pallas import tpu as pltpu
from jax.experimental.pallas import tpu_sc as plsc

N = 10000
E = 320000
D = 128
POS_DIM = 4
FFN_DIM = 256

NC = 2
NS = 16
NW = NC * NS
EPT = E // NW
BLK = 100
NBLK = EPT // BLK
RPT8 = (N // NS) // 8 * 8
RPT_TAIL = N - NS * RPT8
NCHUNK = D // 16
UNROLL = 10

_GDN = lax.GatherDimensionNumbers(
    offset_dims=(), collapsed_slice_dims=(0,), start_index_map=(0,))

_sc_mesh = plsc.VectorSubcoreMesh(core_axis_name="c", subcore_axis_name="s")


@functools.partial(
    pl.kernel,
    out_type=jax.ShapeDtypeStruct((NC, N, D), jnp.float32),
    mesh=_sc_mesh,
    scratch_types=[
        pltpu.VMEM((BLK,), jnp.int32),
        pltpu.VMEM((BLK,), jnp.int32),
        pltpu.VMEM((BLK,), jnp.int32),
        pltpu.VMEM((BLK,), jnp.int32),
        pltpu.VMEM((BLK, D), jnp.float32),
        pltpu.VMEM((BLK, D), jnp.float32),
        pltpu.VMEM((BLK, 16), jnp.float32),
        pltpu.VMEM((BLK, 16), jnp.float32),
        pltpu.VMEM((BLK, 16), jnp.float32),
        pltpu.VMEM((BLK, 16), jnp.float32),
        pltpu.VMEM((BLK, D), jnp.float32),
        pltpu.VMEM((POS_DIM, D), jnp.float32),
        pltpu.VMEM((D,), jnp.float32),
        pltpu.VMEM_SHARED((N, D), jnp.float32),
        pltpu.SemaphoreType.DMA,
        pltpu.SemaphoreType.DMA,
        pltpu.SemaphoreType.DMA,
        pltpu.SemaphoreType.DMA,
    ],
    compiler_params=pltpu.CompilerParams(use_tc_tiling_on_sc=False),
)
def _mp_kernel(x_hbm, pos16_hbm, src_hbm, dst_hbm, wc_hbm, bc_hbm, zeros_hbm,
               out_hbm,
               src0_v, src1_v, dst0_v, dst1_v, xr0_v, xr1_v,
               ps0_v, ps1_v, pd0_v, pd1_v,
               m_v, wc_v, bc_v, agg_sh, sem_ia, sem_ib, sem_ga, sem_gb):
    ci = lax.axis_index("c")
    sid = lax.axis_index("s")
    wid = ci * NS + sid

    pltpu.sync_copy(zeros_hbm.at[pl.ds(sid * RPT8, RPT8)],
                    agg_sh.at[pl.ds(sid * RPT8, RPT8)])

    @pl.when(sid == NS - 1)
    def _zero_tail():
        pltpu.sync_copy(zeros_hbm.at[pl.ds(NS * RPT8, RPT_TAIL)],
                        agg_sh.at[pl.ds(NS * RPT8, RPT_TAIL)])
    pltpu.sync_copy(wc_hbm, wc_v)
    pltpu.sync_copy(bc_hbm, bc_v)
    plsc.subcore_barrier()

    wcs = [[wc_v[i, pl.ds(c * 16, 16)] for c in range(NCHUNK)]
           for i in range(POS_DIM)]
    bcs = [bc_v[pl.ds(c * 16, 16)] for c in range(NCHUNK)]
    lanes = [jnp.full((16, 1), i, jnp.int32) for i in range(POS_DIM)]

    bufs = [
        (src0_v, dst0_v, xr0_v, ps0_v, pd0_v, sem_ia, sem_ga),
        (src1_v, dst1_v, xr1_v, ps1_v, pd1_v, sem_ib, sem_gb),
    ]

    def fire_idx(b, p):
        src_v, dst_v, _, _, _, sem_i, _ = bufs[p]
        pltpu.async_copy(src_hbm.at[wid, b], src_v, sem_i)
        pltpu.async_copy(dst_hbm.at[wid, b], dst_v, sem_i)

    def drain_idx(b, p):
        src_v, dst_v, _, _, _, sem_i, _ = bufs[p]
        pltpu.make_async_copy(src_hbm.at[wid, b], src_v, sem_i).wait()
        pltpu.make_async_copy(dst_hbm.at[wid, b], dst_v, sem_i).wait()

    def fire_gather(p):
        src_v, dst_v, xr, ps, pd, _, sem_g = bufs[p]
        pltpu.async_copy(x_hbm.at[src_v], xr, sem_g)
        pltpu.async_copy(pos16_hbm.at[src_v], ps, sem_g)
        pltpu.async_copy(pos16_hbm.at[dst_v], pd, sem_g)

    def drain_gather(p):
        src_v, dst_v, xr, ps, pd, _, sem_g = bufs[p]
        pltpu.make_async_copy(x_hbm.at[src_v], xr, sem_g).wait()
        pltpu.make_async_copy(pos16_hbm.at[src_v], ps, sem_g).wait()
        pltpu.make_async_copy(pos16_hbm.at[dst_v], pd, sem_g).wait()

    def compute(p):
        src_v, dst_v, xr, ps, pd, _, _ = bufs[p]

        def edge_body(g, _):
            for l in range(UNROLL):
                e = g * UNROLL + l
                dr = jnp.abs(ps[e, :] - pd[e, :])
                d = [lax.gather(dr, lanes[i], _GDN, slice_sizes=(1,),
                                mode=lax.GatherScatterMode.PROMISE_IN_BOUNDS)
                     for i in range(POS_DIM)]
                for c in range(NCHUNK):
                    acc = xr[e, pl.ds(c * 16, 16)] + bcs[c]
                    acc = acc + (d[0] * wcs[0][c] + d[1] * wcs[1][c])
                    acc = acc + (d[2] * wcs[2][c] + d[3] * wcs[3][c])
                    m_v[e, pl.ds(c * 16, 16)] = jnp.maximum(acc, 0.0)
            return 0

        lax.fori_loop(0, BLK // UNROLL, edge_body, 0)
        pltpu.sync_copy(m_v, agg_sh.at[dst_v], add=True)

    fire_idx(0, 0)
    drain_idx(0, 0)
    fire_gather(0)
    fire_idx(1, 1)

    def two_blocks(t, _):
        b1 = 2 * t + 1
        drain_idx(b1, 1)
        fire_gather(1)
        drain_gather(0)
        compute(0)

        @pl.when(b1 + 1 < NBLK)
        def _prep_even():
            fire_idx(b1 + 1, 0)
            drain_idx(b1 + 1, 0)
            fire_gather(0)

        drain_gather(1)
        compute(1)

        @pl.when(b1 + 2 < NBLK)
        def _prep_odd():
            fire_idx(b1 + 2, 1)

        return 0

    lax.fori_loop(0, NBLK // 2, two_blocks, 0)
    plsc.subcore_barrier()
    pltpu.sync_copy(agg_sh.at[pl.ds(sid * RPT8, RPT8)],
                    out_hbm.at[ci, pl.ds(sid * RPT8, RPT8)])

    @pl.when(sid == NS - 1)
    def _write_tail():
        pltpu.sync_copy(agg_sh.at[pl.ds(NS * RPT8, RPT_TAIL)],
                        out_hbm.at[ci, pl.ds(NS * RPT8, RPT_TAIL)])


_ROWS = 1000


def _dense_body(scale_ref, x_ref, a0_ref, a1_ref, pos_ref, pad_ref,
                w1_ref, b1_ref, w2_ref, b2_ref, g1_ref, be1_ref,
                wf1_ref, bf1_ref, wf2_ref, bf2_ref, g2_ref, be2_ref,
                offw_ref, offb_ref,
                xo_ref, poso_ref, pado_ref):
    x = x_ref[...]
    h = scale_ref[0, 0] * x + (a0_ref[...] + a1_ref[...])
    h = jnp.maximum(jnp.dot(h, w1_ref[...], preferred_element_type=jnp.float32)
                    + b1_ref[...], 0.0)
    h = jnp.dot(h, w2_ref[...], preferred_element_type=jnp.float32) + b2_ref[...]
    h = x + h
    mu = jnp.mean(h, axis=-1, keepdims=True)
    var = jnp.mean(jnp.square(h - mu), axis=-1, keepdims=True)
    h = (h - mu) * lax.rsqrt(var + 1e-5) * g1_ref[...] + be1_ref[...]
    f = jnp.maximum(jnp.dot(h, wf1_ref[...], preferred_element_type=jnp.float32)
                    + bf1_ref[...], 0.0)
    f = jnp.dot(f, wf2_ref[...], preferred_element_type=jnp.float32) + bf2_ref[...]
    h = h + f
    mu = jnp.mean(h, axis=-1, keepdims=True)
    var = jnp.mean(jnp.square(h - mu), axis=-1, keepdims=True)
    h = (h - mu) * lax.rsqrt(var + 1e-5) * g2_ref[...] + be2_ref[...]
    xo_ref[...] = h
    off = jnp.dot(h, offw_ref[...], preferred_element_type=jnp.float32) + offb_ref[...]
    poso_ref[...] = pos_ref[...] + off
    pado_ref[...] = pad_ref[...] + off


def _dense_layer(scale, x, agg, pos, pad, p, off_w, off_b):
    row_spec = pl.BlockSpec((_ROWS, D), lambda i: (i, 0))
    pos_spec = pl.BlockSpec((_ROWS, POS_DIM), lambda i: (i, 0))

    def full(a):
        return pl.BlockSpec(a.shape, lambda i: tuple(0 for _ in a.shape))

    w1 = p["W1"]
    b1 = p["b1"].reshape(1, D)
    w2 = p["W2"]
    b2 = p["b2"].reshape(1, D)
    g1 = p["ln1_g"].reshape(1, D)
    be1 = p["ln1_b"].reshape(1, D)
    wf1 = p["Wf1"]
    bf1 = p["bf1"].reshape(1, FFN_DIM)
    wf2 = p["Wf2"]
    bf2 = p["bf2"].reshape(1, D)
    g2 = p["ln2_g"].reshape(1, D)
    be2 = p["ln2_b"].reshape(1, D)
    offb = off_b.reshape(1, POS_DIM)
    scale2 = scale.reshape(1, 1)

    return pl.pallas_call(
        _dense_body,
        grid=(N // _ROWS,),
        in_specs=[
            full(scale2), row_spec, row_spec, row_spec, pos_spec, pos_spec,
            full(w1), full(b1), full(w2), full(b2), full(g1), full(be1),
            full(wf1), full(bf1), full(wf2), full(bf2), full(g2), full(be2),
            full(off_w), full(offb),
        ],
        out_specs=[row_spec, pos_spec, pos_spec],
        out_shape=[
            jax.ShapeDtypeStruct((N, D), jnp.float32),
            jax.ShapeDtypeStruct((N, POS_DIM), jnp.float32),
            jax.ShapeDtypeStruct((N, POS_DIM), jnp.float32),
        ],
    )(scale2, x, agg[0], agg[1], pos, pad,
      w1, b1, w2, b2, g1, be1, wf1, bf1, wf2, bf2, g2, be2, off_w, offb)


def kernel(x, pos_enc, params, edge_index, node_indices):
    src = edge_index[0].astype(jnp.int32)
    dst = edge_index[1].astype(jnp.int32)
    we = params["edge_embed_W"]
    be = params["edge_embed_b"]
    off_w = params["off_W"]
    off_b = params["off_b"]
    zeros = jnp.zeros((N, D), jnp.float32)

    pos = pos_enc
    pad = jnp.zeros((N, POS_DIM), jnp.float32)
    for p in params["layers"]:
        wc = we @ p["edge_W"]
        bc = be @ p["edge_W"] + p["edge_b"]
        pos16 = jnp.pad(pos, ((0, 0), (0, 16 - POS_DIM)))
        agg = _mp_kernel(x, pos16, src.reshape(NW, NBLK, BLK),
                         dst.reshape(NW, NBLK, BLK), wc, bc, zeros)
        scale = (1.0 + p["eps"]).astype(jnp.float32)
        x, pos, pad = _dense_layer(scale, x, agg, pos, pad, p, off_w, off_b)
    return jnp.concatenate((pad, x), axis=1)


# speedup vs baseline: 6.7678x; 1.8019x over previous
import functools

import jax
import jax.numpy as jnp
from jax import lax
from jax.experimental import pallas as pl
from jax.experimental.pallas import tpu as pltpu
from jax.experimental.pallas import tpu_sc as plsc

N = 10000
E = 320000
D = 128
POS_DIM = 4
FFN_DIM = 256

NC = 2
NS = 16
NW = NC * NS
EPT = E // NW
BLK = 100
NBLK = EPT // BLK
RPT8 = (N // NS) // 8 * 8
RPT_TAIL = N - NS * RPT8
NCHUNK = D // 16
UNROLL = 10

_GDN = lax.GatherDimensionNumbers(
    offset_dims=(), collapsed_slice_dims=(0,), start_index_map=(0,))

_sc_mesh = plsc.VectorSubcoreMesh(core_axis_name="c", subcore_axis_name="s")


@functools.partial(
    pl.kernel,
    out_type=jax.ShapeDtypeStruct((NC, N, D), jnp.float32),
    mesh=_sc_mesh,
    scratch_types=[
        pltpu.VMEM((BLK,), jnp.int32),
        pltpu.VMEM((BLK,), jnp.int32),
        pltpu.VMEM((BLK,), jnp.int32),
        pltpu.VMEM((BLK,), jnp.int32),
        pltpu.VMEM((BLK, D), jnp.float32),
        pltpu.VMEM((BLK, D), jnp.float32),
        pltpu.VMEM((BLK, 16), jnp.float32),
        pltpu.VMEM((BLK, 16), jnp.float32),
        pltpu.VMEM((BLK, 16), jnp.float32),
        pltpu.VMEM((BLK, 16), jnp.float32),
        pltpu.VMEM((BLK, D), jnp.float32),
        pltpu.VMEM((POS_DIM, D), jnp.float32),
        pltpu.VMEM((D,), jnp.float32),
        pltpu.VMEM_SHARED((N, D), jnp.float32),
        pltpu.SemaphoreType.DMA,
        pltpu.SemaphoreType.DMA,
        pltpu.SemaphoreType.DMA,
        pltpu.SemaphoreType.DMA,
    ],
    compiler_params=pltpu.CompilerParams(use_tc_tiling_on_sc=False),
)
def _mp_kernel(x_hbm, pos16_hbm, src_hbm, dst_hbm, wc_hbm, bc_hbm, zeros_hbm,
               out_hbm,
               src0_v, src1_v, dst0_v, dst1_v, xr0_v, xr1_v,
               ps0_v, ps1_v, pd0_v, pd1_v,
               m_v, wc_v, bc_v, agg_sh, sem_ia, sem_ib, sem_ga, sem_gb):
    ci = lax.axis_index("c")
    sid = lax.axis_index("s")
    wid = ci * NS + sid

    pltpu.sync_copy(zeros_hbm.at[pl.ds(sid * RPT8, RPT8)],
                    agg_sh.at[pl.ds(sid * RPT8, RPT8)])

    @pl.when(sid == NS - 1)
    def _zero_tail():
        pltpu.sync_copy(zeros_hbm.at[pl.ds(NS * RPT8, RPT_TAIL)],
                        agg_sh.at[pl.ds(NS * RPT8, RPT_TAIL)])
    pltpu.sync_copy(wc_hbm, wc_v)
    pltpu.sync_copy(bc_hbm, bc_v)
    plsc.subcore_barrier()

    wcs = [[wc_v[i, pl.ds(c * 16, 16)] for c in range(NCHUNK)]
           for i in range(POS_DIM)]
    bcs = [bc_v[pl.ds(c * 16, 16)] for c in range(NCHUNK)]
    lanes = [jnp.full((16, 1), i, jnp.int32) for i in range(POS_DIM)]

    bufs = [
        (src0_v, dst0_v, xr0_v, ps0_v, pd0_v, sem_ia, sem_ga),
        (src1_v, dst1_v, xr1_v, ps1_v, pd1_v, sem_ib, sem_gb),
    ]

    def fire_idx(b, p):
        src_v, dst_v, _, _, _, sem_i, _ = bufs[p]
        pltpu.async_copy(src_hbm.at[wid, b], src_v, sem_i)
        pltpu.async_copy(dst_hbm.at[wid, b], dst_v, sem_i)

    def drain_idx(b, p):
        src_v, dst_v, _, _, _, sem_i, _ = bufs[p]
        pltpu.make_async_copy(src_hbm.at[wid, b], src_v, sem_i).wait()
        pltpu.make_async_copy(dst_hbm.at[wid, b], dst_v, sem_i).wait()

    def fire_gather(p):
        src_v, dst_v, xr, ps, pd, _, sem_g = bufs[p]
        pltpu.async_copy(x_hbm.at[src_v], xr, sem_g)
        pltpu.async_copy(pos16_hbm.at[src_v], ps, sem_g)
        pltpu.async_copy(pos16_hbm.at[dst_v], pd, sem_g)

    def drain_gather(p):
        src_v, dst_v, xr, ps, pd, _, sem_g = bufs[p]
        pltpu.make_async_copy(x_hbm.at[src_v], xr, sem_g).wait()
        pltpu.make_async_copy(pos16_hbm.at[src_v], ps, sem_g).wait()
        pltpu.make_async_copy(pos16_hbm.at[dst_v], pd, sem_g).wait()

    def compute(p):
        src_v, dst_v, xr, ps, pd, _, _ = bufs[p]

        @plsc.parallel_loop(0, BLK, step=1, unroll=UNROLL)
        def edge_body(e):
            dr = jnp.abs(ps[e, :] - pd[e, :])
            d = [lax.gather(dr, lanes[i], _GDN, slice_sizes=(1,),
                            mode=lax.GatherScatterMode.PROMISE_IN_BOUNDS)
                 for i in range(POS_DIM)]
            for c in range(NCHUNK):
                acc = xr[e, pl.ds(c * 16, 16)] + bcs[c]
                acc = acc + (d[0] * wcs[0][c] + d[1] * wcs[1][c])
                acc = acc + (d[2] * wcs[2][c] + d[3] * wcs[3][c])
                m_v[e, pl.ds(c * 16, 16)] = jnp.maximum(acc, 0.0)

        pltpu.sync_copy(m_v, agg_sh.at[dst_v], add=True)

    fire_idx(0, 0)
    drain_idx(0, 0)
    fire_gather(0)
    fire_idx(1, 1)

    def two_blocks(t, _):
        b1 = 2 * t + 1
        drain_idx(b1, 1)
        fire_gather(1)
        drain_gather(0)
        compute(0)

        @pl.when(b1 + 1 < NBLK)
        def _prep_even():
            fire_idx(b1 + 1, 0)
            drain_idx(b1 + 1, 0)
            fire_gather(0)

        drain_gather(1)
        compute(1)

        @pl.when(b1 + 2 < NBLK)
        def _prep_odd():
            fire_idx(b1 + 2, 1)

        return 0

    lax.fori_loop(0, NBLK // 2, two_blocks, 0)
    plsc.subcore_barrier()
    pltpu.sync_copy(agg_sh.at[pl.ds(sid * RPT8, RPT8)],
                    out_hbm.at[ci, pl.ds(sid * RPT8, RPT8)])

    @pl.when(sid == NS - 1)
    def _write_tail():
        pltpu.sync_copy(agg_sh.at[pl.ds(NS * RPT8, RPT_TAIL)],
                        out_hbm.at[ci, pl.ds(NS * RPT8, RPT_TAIL)])


_ROWS = 1000


def _dense_body(scale_ref, x_ref, a0_ref, a1_ref, pos_ref, pad_ref,
                w1_ref, b1_ref, w2_ref, b2_ref, g1_ref, be1_ref,
                wf1_ref, bf1_ref, wf2_ref, bf2_ref, g2_ref, be2_ref,
                offw_ref, offb_ref,
                xo_ref, poso_ref, pado_ref):
    x = x_ref[...]
    h = scale_ref[0, 0] * x + (a0_ref[...] + a1_ref[...])
    h = jnp.maximum(jnp.dot(h, w1_ref[...], preferred_element_type=jnp.float32)
                    + b1_ref[...], 0.0)
    h = jnp.dot(h, w2_ref[...], preferred_element_type=jnp.float32) + b2_ref[...]
    h = x + h
    mu = jnp.mean(h, axis=-1, keepdims=True)
    var = jnp.mean(jnp.square(h - mu), axis=-1, keepdims=True)
    h = (h - mu) * lax.rsqrt(var + 1e-5) * g1_ref[...] + be1_ref[...]
    f = jnp.maximum(jnp.dot(h, wf1_ref[...], preferred_element_type=jnp.float32)
                    + bf1_ref[...], 0.0)
    f = jnp.dot(f, wf2_ref[...], preferred_element_type=jnp.float32) + bf2_ref[...]
    h = h + f
    mu = jnp.mean(h, axis=-1, keepdims=True)
    var = jnp.mean(jnp.square(h - mu), axis=-1, keepdims=True)
    h = (h - mu) * lax.rsqrt(var + 1e-5) * g2_ref[...] + be2_ref[...]
    xo_ref[...] = h
    off = jnp.dot(h, offw_ref[...], preferred_element_type=jnp.float32) + offb_ref[...]
    poso_ref[...] = pos_ref[...] + off
    pado_ref[...] = pad_ref[...] + off


def _dense_layer(scale, x, agg, pos, pad, p, off_w, off_b):
    row_spec = pl.BlockSpec((_ROWS, D), lambda i: (i, 0))
    pos_spec = pl.BlockSpec((_ROWS, POS_DIM), lambda i: (i, 0))

    def full(a):
        return pl.BlockSpec(a.shape, lambda i: tuple(0 for _ in a.shape))

    w1 = p["W1"]
    b1 = p["b1"].reshape(1, D)
    w2 = p["W2"]
    b2 = p["b2"].reshape(1, D)
    g1 = p["ln1_g"].reshape(1, D)
    be1 = p["ln1_b"].reshape(1, D)
    wf1 = p["Wf1"]
    bf1 = p["bf1"].reshape(1, FFN_DIM)
    wf2 = p["Wf2"]
    bf2 = p["bf2"].reshape(1, D)
    g2 = p["ln2_g"].reshape(1, D)
    be2 = p["ln2_b"].reshape(1, D)
    offb = off_b.reshape(1, POS_DIM)
    scale2 = scale.reshape(1, 1)

    return pl.pallas_call(
        _dense_body,
        grid=(N // _ROWS,),
        in_specs=[
            full(scale2), row_spec, row_spec, row_spec, pos_spec, pos_spec,
            full(w1), full(b1), full(w2), full(b2), full(g1), full(be1),
            full(wf1), full(bf1), full(wf2), full(bf2), full(g2), full(be2),
            full(off_w), full(offb),
        ],
        out_specs=[row_spec, pos_spec, pos_spec],
        out_shape=[
            jax.ShapeDtypeStruct((N, D), jnp.float32),
            jax.ShapeDtypeStruct((N, POS_DIM), jnp.float32),
            jax.ShapeDtypeStruct((N, POS_DIM), jnp.float32),
        ],
    )(scale2, x, agg[0], agg[1], pos, pad,
      w1, b1, w2, b2, g1, be1, wf1, bf1, wf2, bf2, g2, be2, off_w, offb)


def kernel(x, pos_enc, params, edge_index, node_indices):
    src = edge_index[0].astype(jnp.int32)
    dst = edge_index[1].astype(jnp.int32)
    we = params["edge_embed_W"]
    be = params["edge_embed_b"]
    off_w = params["off_W"]
    off_b = params["off_b"]
    zeros = jnp.zeros((N, D), jnp.float32)

    pos = pos_enc
    pad = jnp.zeros((N, POS_DIM), jnp.float32)
    for p in params["layers"]:
        wc = we @ p["edge_W"]
        bc = be @ p["edge_W"] + p["edge_b"]
        pos16 = jnp.pad(pos, ((0, 0), (0, 16 - POS_DIM)))
        agg = _mp_kernel(x, pos16, src.reshape(NW, NBLK, BLK),
                         dst.reshape(NW, NBLK, BLK), wc, bc, zeros)
        scale = (1.0 + p["eps"]).astype(jnp.float32)
        x, pos, pad = _dense_layer(scale, x, agg, pos, pad, p, off_w, off_b)
    return jnp.concatenate((pad, x), axis=1)


# speedup vs baseline: 8.4582x; 1.2498x over previous
import functools

import jax
import jax.numpy as jnp
from jax import lax
from jax.experimental import pallas as pl
from jax.experimental.pallas import tpu as pltpu
from jax.experimental.pallas import tpu_sc as plsc

N = 10000
E = 320000
D = 128
POS_DIM = 4
FFN_DIM = 256

NC = 2
NS = 16
NW = NC * NS
EPT = E // NW
BLK = 80
NBLK = EPT // BLK
RPT8 = (N // NS) // 8 * 8
RPT_TAIL = N - NS * RPT8
NCHUNK = D // 16
UNROLL = 10

_GDN = lax.GatherDimensionNumbers(
    offset_dims=(), collapsed_slice_dims=(0,), start_index_map=(0,))

_sc_mesh = plsc.VectorSubcoreMesh(core_axis_name="c", subcore_axis_name="s")


@functools.partial(
    pl.kernel,
    out_type=jax.ShapeDtypeStruct((NC, N, D), jnp.float32),
    mesh=_sc_mesh,
    scratch_types=[
        pltpu.VMEM((BLK,), jnp.int32),
        pltpu.VMEM((BLK,), jnp.int32),
        pltpu.VMEM((BLK,), jnp.int32),
        pltpu.VMEM((BLK,), jnp.int32),
        pltpu.VMEM((BLK, D), jnp.float32),
        pltpu.VMEM((BLK, D), jnp.float32),
        pltpu.VMEM((BLK, 16), jnp.float32),
        pltpu.VMEM((BLK, 16), jnp.float32),
        pltpu.VMEM((BLK, 16), jnp.float32),
        pltpu.VMEM((BLK, 16), jnp.float32),
        pltpu.VMEM((BLK,), jnp.int32),
        pltpu.VMEM((BLK,), jnp.int32),
        pltpu.VMEM((BLK, D), jnp.float32),
        pltpu.VMEM((BLK, D), jnp.float32),
        pltpu.VMEM((POS_DIM, D), jnp.float32),
        pltpu.VMEM((D,), jnp.float32),
        pltpu.VMEM_SHARED((N, D), jnp.float32),
        pltpu.SemaphoreType.DMA,
        pltpu.SemaphoreType.DMA,
        pltpu.SemaphoreType.DMA,
        pltpu.SemaphoreType.DMA,
        pltpu.SemaphoreType.DMA,
        pltpu.SemaphoreType.DMA,
    ],
    compiler_params=pltpu.CompilerParams(use_tc_tiling_on_sc=False),
)
def _mp_kernel(x_hbm, pos16_hbm, src_hbm, dst_hbm, wc_hbm, bc_hbm, zeros_hbm,
               out_hbm,
               src0_v, src1_v, dst0_v, dst1_v, xr0_v, xr1_v,
               ps0_v, ps1_v, pd0_v, pd1_v, sd0_v, sd1_v,
               m0_v, m1_v, wc_v, bc_v, agg_sh,
               sem_ia, sem_ib, sem_ga, sem_gb, sem_sa, sem_sb):
    ci = lax.axis_index("c")
    sid = lax.axis_index("s")
    wid = ci * NS + sid

    pltpu.sync_copy(zeros_hbm.at[pl.ds(sid * RPT8, RPT8)],
                    agg_sh.at[pl.ds(sid * RPT8, RPT8)])

    @pl.when(sid == NS - 1)
    def _zero_tail():
        pltpu.sync_copy(zeros_hbm.at[pl.ds(NS * RPT8, RPT_TAIL)],
                        agg_sh.at[pl.ds(NS * RPT8, RPT_TAIL)])
    pltpu.sync_copy(wc_hbm, wc_v)
    pltpu.sync_copy(bc_hbm, bc_v)
    plsc.subcore_barrier()

    wcs = [[wc_v[i, pl.ds(c * 16, 16)] for c in range(NCHUNK)]
           for i in range(POS_DIM)]
    bcs = [bc_v[pl.ds(c * 16, 16)] for c in range(NCHUNK)]
    lanes = [jnp.full((16, 1), i, jnp.int32) for i in range(POS_DIM)]

    bufs = [
        (src0_v, dst0_v, xr0_v, ps0_v, pd0_v, sd0_v, m0_v,
         sem_ia, sem_ga, sem_sa),
        (src1_v, dst1_v, xr1_v, ps1_v, pd1_v, sd1_v, m1_v,
         sem_ib, sem_gb, sem_sb),
    ]

    def fire_idx(b, p):
        src_v, dst_v = bufs[p][0], bufs[p][1]
        sem_i = bufs[p][7]
        pltpu.async_copy(src_hbm.at[wid, b], src_v, sem_i)
        pltpu.async_copy(dst_hbm.at[wid, b], dst_v, sem_i)

    def drain_idx(b, p):
        src_v, dst_v = bufs[p][0], bufs[p][1]
        sem_i = bufs[p][7]
        pltpu.make_async_copy(src_hbm.at[wid, b], src_v, sem_i).wait()
        pltpu.make_async_copy(dst_hbm.at[wid, b], dst_v, sem_i).wait()

    def fire_gather(p):
        src_v, dst_v, xr, ps, pd = bufs[p][:5]
        sem_g = bufs[p][8]
        pltpu.async_copy(x_hbm.at[src_v], xr, sem_g)
        pltpu.async_copy(pos16_hbm.at[src_v], ps, sem_g)
        pltpu.async_copy(pos16_hbm.at[dst_v], pd, sem_g)

    def drain_gather(p):
        src_v, dst_v, xr, ps, pd = bufs[p][:5]
        sem_g = bufs[p][8]
        pltpu.make_async_copy(x_hbm.at[src_v], xr, sem_g).wait()
        pltpu.make_async_copy(pos16_hbm.at[src_v], ps, sem_g).wait()
        pltpu.make_async_copy(pos16_hbm.at[dst_v], pd, sem_g).wait()

    def shadow(p):
        dst_v, sd_v = bufs[p][1], bufs[p][5]
        for k in range(BLK // 16):
            sd_v[pl.ds(k * 16, 16)] = dst_v[pl.ds(k * 16, 16)]

    def compute(p):
        dst_v, xr, ps, pd, sd_v, m_v = bufs[p][1:7]
        sem_s = bufs[p][9]

        @plsc.parallel_loop(0, BLK, step=1, unroll=UNROLL)
        def edge_body(e):
            dr = jnp.abs(ps[e, :] - pd[e, :])
            d = [lax.gather(dr, lanes[i], _GDN, slice_sizes=(1,),
                            mode=lax.GatherScatterMode.PROMISE_IN_BOUNDS)
                 for i in range(POS_DIM)]
            for c in range(NCHUNK):
                acc = xr[e, pl.ds(c * 16, 16)] + bcs[c]
                acc = acc + (d[0] * wcs[0][c] + d[1] * wcs[1][c])
                acc = acc + (d[2] * wcs[2][c] + d[3] * wcs[3][c])
                m_v[e, pl.ds(c * 16, 16)] = jnp.maximum(acc, 0.0)

        pltpu.async_copy(m_v, agg_sh.at[sd_v], sem_s, add=True)

    def drain_scatter(p):
        sd_v, m_v = bufs[p][5], bufs[p][6]
        sem_s = bufs[p][9]
        pltpu.make_async_copy(m_v, agg_sh.at[sd_v], sem_s).wait()

    fire_idx(0, 0)
    drain_idx(0, 0)
    fire_gather(0)
    fire_idx(1, 1)
    drain_idx(1, 1)
    fire_gather(1)

    NPAIR = NBLK // 2

    def two_blocks(t, _):
        b0 = 2 * t
        b1 = b0 + 1

        @pl.when(t > 0)
        def _ds0():
            drain_scatter(0)

        drain_gather(0)
        shadow(0)
        fire_idx(b0 + 2, 0)
        compute(0)
        drain_idx(b0 + 2, 0)
        fire_gather(0)

        @pl.when(t > 0)
        def _ds1():
            drain_scatter(1)

        drain_gather(1)
        shadow(1)

        @pl.when(t < NPAIR - 1)
        def _prep1():
            fire_idx(b1 + 2, 1)

        compute(1)

        @pl.when(t < NPAIR - 1)
        def _fin1():
            drain_idx(b1 + 2, 1)
            fire_gather(1)

        return 0

    lax.fori_loop(0, NPAIR, two_blocks, 0)
    drain_scatter(0)
    drain_gather(0)
    shadow(0)
    compute(0)
    drain_scatter(1)
    drain_scatter(0)
    plsc.subcore_barrier()
    pltpu.sync_copy(agg_sh.at[pl.ds(sid * RPT8, RPT8)],
                    out_hbm.at[ci, pl.ds(sid * RPT8, RPT8)])

    @pl.when(sid == NS - 1)
    def _write_tail():
        pltpu.sync_copy(agg_sh.at[pl.ds(NS * RPT8, RPT_TAIL)],
                        out_hbm.at[ci, pl.ds(NS * RPT8, RPT_TAIL)])


_ROWS = 1000


def _dense_body(scale_ref, x_ref, a0_ref, a1_ref, pos_ref, pad_ref,
                w1_ref, b1_ref, w2_ref, b2_ref, g1_ref, be1_ref,
                wf1_ref, bf1_ref, wf2_ref, bf2_ref, g2_ref, be2_ref,
                offw_ref, offb_ref,
                xo_ref, poso_ref, pado_ref):
    x = x_ref[...]
    h = scale_ref[0, 0] * x + (a0_ref[...] + a1_ref[...])
    h = jnp.maximum(jnp.dot(h, w1_ref[...], preferred_element_type=jnp.float32)
                    + b1_ref[...], 0.0)
    h = jnp.dot(h, w2_ref[...], preferred_element_type=jnp.float32) + b2_ref[...]
    h = x + h
    mu = jnp.mean(h, axis=-1, keepdims=True)
    var = jnp.mean(jnp.square(h - mu), axis=-1, keepdims=True)
    h = (h - mu) * lax.rsqrt(var + 1e-5) * g1_ref[...] + be1_ref[...]
    f = jnp.maximum(jnp.dot(h, wf1_ref[...], preferred_element_type=jnp.float32)
                    + bf1_ref[...], 0.0)
    f = jnp.dot(f, wf2_ref[...], preferred_element_type=jnp.float32) + bf2_ref[...]
    h = h + f
    mu = jnp.mean(h, axis=-1, keepdims=True)
    var = jnp.mean(jnp.square(h - mu), axis=-1, keepdims=True)
    h = (h - mu) * lax.rsqrt(var + 1e-5) * g2_ref[...] + be2_ref[...]
    xo_ref[...] = h
    off = jnp.dot(h, offw_ref[...], preferred_element_type=jnp.float32) + offb_ref[...]
    poso_ref[...] = pos_ref[...] + off
    pado_ref[...] = pad_ref[...] + off


def _dense_layer(scale, x, agg, pos, pad, p, off_w, off_b):
    row_spec = pl.BlockSpec((_ROWS, D), lambda i: (i, 0))
    pos_spec = pl.BlockSpec((_ROWS, POS_DIM), lambda i: (i, 0))

    def full(a):
        return pl.BlockSpec(a.shape, lambda i: tuple(0 for _ in a.shape))

    w1 = p["W1"]
    b1 = p["b1"].reshape(1, D)
    w2 = p["W2"]
    b2 = p["b2"].reshape(1, D)
    g1 = p["ln1_g"].reshape(1, D)
    be1 = p["ln1_b"].reshape(1, D)
    wf1 = p["Wf1"]
    bf1 = p["bf1"].reshape(1, FFN_DIM)
    wf2 = p["Wf2"]
    bf2 = p["bf2"].reshape(1, D)
    g2 = p["ln2_g"].reshape(1, D)
    be2 = p["ln2_b"].reshape(1, D)
    offb = off_b.reshape(1, POS_DIM)
    scale2 = scale.reshape(1, 1)

    return pl.pallas_call(
        _dense_body,
        grid=(N // _ROWS,),
        in_specs=[
            full(scale2), row_spec, row_spec, row_spec, pos_spec, pos_spec,
            full(w1), full(b1), full(w2), full(b2), full(g1), full(be1),
            full(wf1), full(bf1), full(wf2), full(bf2), full(g2), full(be2),
            full(off_w), full(offb),
        ],
        out_specs=[row_spec, pos_spec, pos_spec],
        out_shape=[
            jax.ShapeDtypeStruct((N, D), jnp.float32),
            jax.ShapeDtypeStruct((N, POS_DIM), jnp.float32),
            jax.ShapeDtypeStruct((N, POS_DIM), jnp.float32),
        ],
    )(scale2, x, agg[0], agg[1], pos, pad,
      w1, b1, w2, b2, g1, be1, wf1, bf1, wf2, bf2, g2, be2, off_w, offb)


def kernel(x, pos_enc, params, edge_index, node_indices):
    src = edge_index[0].astype(jnp.int32)
    dst = edge_index[1].astype(jnp.int32)
    we = params["edge_embed_W"]
    be = params["edge_embed_b"]
    off_w = params["off_W"]
    off_b = params["off_b"]
    zeros = jnp.zeros((N, D), jnp.float32)

    pos = pos_enc
    pad = jnp.zeros((N, POS_DIM), jnp.float32)
    for p in params["layers"]:
        wc = we @ p["edge_W"]
        bc = be @ p["edge_W"] + p["edge_b"]
        pos16 = jnp.pad(pos, ((0, 0), (0, 16 - POS_DIM)))
        agg = _mp_kernel(x, pos16, src.reshape(NW, NBLK, BLK),
                         dst.reshape(NW, NBLK, BLK), wc, bc, zeros)
        scale = (1.0 + p["eps"]).astype(jnp.float32)
        x, pos, pad = _dense_layer(scale, x, agg, pos, pad, p, off_w, off_b)
    return jnp.concatenate((pad, x), axis=1)
